# Initial kernel scaffold; baseline (speedup 1.0000x reference)
import jax
import jax.numpy as jnp
from jax.experimental import pallas as pl


def kernel(x, pe_table):
    raise NotImplementedError("write your pallas kernel here")


# speedup vs baseline: 1.3712x; 1.3712x over previous
import math

import jax
import jax.numpy as jnp
from jax.experimental import pallas as pl
from jax.experimental.pallas import tpu as pltpu

_D = 768
_L = 8192
_B = 4
_BL = 1024
_SCALE = math.sqrt(_D)


def _pe_add_body(x_ref, pe_ref, o_ref):
    o_ref[...] = x_ref[...] + pe_ref[...] * _SCALE


def kernel(x, pe_table):
    grid = (_B, _L // _BL)
    return pl.pallas_call(
        _pe_add_body,
        grid=grid,
        in_specs=[
            pl.BlockSpec((1, _BL, _D), lambda b, l: (b, l, 0)),
            pl.BlockSpec((_BL, _D), lambda b, l: (l, 0)),
        ],
        out_specs=pl.BlockSpec((1, _BL, _D), lambda b, l: (b, l, 0)),
        out_shape=jax.ShapeDtypeStruct((_B, _L, _D), jnp.float32),
        compiler_params=pltpu.CompilerParams(
            dimension_semantics=("parallel", "arbitrary"),
        ),
    )(x, pe_table)


# speedup vs baseline: 1.6772x; 1.2232x over previous
import math

import jax
import jax.numpy as jnp
from jax.experimental import pallas as pl
from jax.experimental.pallas import tpu as pltpu

_D = 768
_L = 8192
_B = 4
_BL = 1024
_SCALE = math.sqrt(_D)


def _pe_add_body(x_ref, pe_ref, o_ref):
    o_ref[...] = x_ref[...] + pe_ref[...] * _SCALE


def kernel(x, pe_table):
    grid = (_L // _BL, _B)
    return pl.pallas_call(
        _pe_add_body,
        grid=grid,
        in_specs=[
            pl.BlockSpec((1, _BL, _D), lambda l, b: (b, l, 0)),
            pl.BlockSpec((_BL, _D), lambda l, b: (l, 0)),
        ],
        out_specs=pl.BlockSpec((1, _BL, _D), lambda l, b: (b, l, 0)),
        out_shape=jax.ShapeDtypeStruct((_B, _L, _D), jnp.float32),
        compiler_params=pltpu.CompilerParams(
            dimension_semantics=("arbitrary", "arbitrary"),
        ),
    )(x, pe_table)


# speedup vs baseline: 1.7919x; 1.0684x over previous
import math

import jax
import jax.numpy as jnp
from jax.experimental import pallas as pl
from jax.experimental.pallas import tpu as pltpu

_D = 768
_L = 8192
_B = 4
_BL = 2048
_SCALE = math.sqrt(_D)


def _pe_add_body(x_ref, pe_ref, o_ref):
    o_ref[...] = x_ref[...] + pe_ref[...] * _SCALE


def kernel(x, pe_table):
    grid = (_L // _BL, _B)
    return pl.pallas_call(
        _pe_add_body,
        grid=grid,
        in_specs=[
            pl.BlockSpec((1, _BL, _D), lambda l, b: (b, l, 0)),
            pl.BlockSpec((_BL, _D), lambda l, b: (l, 0)),
        ],
        out_specs=pl.BlockSpec((1, _BL, _D), lambda l, b: (b, l, 0)),
        out_shape=jax.ShapeDtypeStruct((_B, _L, _D), jnp.float32),
        compiler_params=pltpu.CompilerParams(
            dimension_semantics=("arbitrary", "arbitrary"),
        ),
    )(x, pe_table)
